# baseline (speedup 1.0000x reference)
import functools

import jax
import jax.numpy as jnp
from jax import lax
from jax.experimental import pallas as pl
from jax.experimental.pallas import tpu as pltpu
from jax.experimental.pallas import tpu_sc as plsc

_B = 16384
_D = 32
_NC = 2
_NS = 16
_NW = _NC * _NS
_BPW = _B // _NW
_CH = 128
_NCHUNK = _BPW // _CH

_mesh = plsc.VectorSubcoreMesh(core_axis_name="c", subcore_axis_name="s")


@functools.partial(
    pl.kernel,
    out_type=jax.ShapeDtypeStruct((_B, 3 * _D), jnp.float32),
    mesh=_mesh,
    scratch_types=[
        pltpu.VMEM((_NCHUNK, _CH), jnp.int32),
        pltpu.VMEM((_BPW, _D), jnp.float32),
        pltpu.SemaphoreType.DMA,
    ],
    compiler_params=pltpu.CompilerParams(use_tc_tiling_on_sc=False),
)
def _gather_concat(p_idx, b_idx, t_idx, w_p, w_b, w_t, out, idx_v, rows_v, sem):
    wid = lax.axis_index("s") * _NC + lax.axis_index("c")
    for f, (ih, wh) in enumerate(((p_idx, w_p), (b_idx, w_b), (t_idx, w_t))):
        pltpu.sync_copy(ih.at[pl.ds(wid * _NCHUNK, _NCHUNK)], idx_v)
        for j in range(_NCHUNK):
            pltpu.async_copy(
                wh.at[idx_v.at[j]], rows_v.at[pl.ds(j * _CH, _CH)], sem
            ).wait()
        pltpu.sync_copy(
            rows_v, out.at[pl.ds(wid * _BPW, _BPW), pl.ds(f * _D, _D)]
        )


def kernel(positions, bet_sizing_id, topology, W_positions, W_bet_sizing_id, W_topology):
    p = positions.reshape(_NW * _NCHUNK, _CH)
    b = bet_sizing_id.reshape(_NW * _NCHUNK, _CH)
    t = topology.reshape(_NW * _NCHUNK, _CH)
    return _gather_concat(p, b, t, W_positions, W_bet_sizing_id, W_topology)
